# Initial kernel scaffold; baseline (speedup 1.0000x reference)
import jax
import jax.numpy as jnp
from jax.experimental import pallas as pl


def kernel(x, params, edge_index, batch, edge_mask):
    raise NotImplementedError("write your pallas kernel here")


# speedup vs baseline: 2.9381x; 2.9381x over previous
import functools

import jax
import jax.numpy as jnp
from jax import lax
from jax.experimental import pallas as pl
from jax.experimental.pallas import tpu as pltpu
from jax.experimental.pallas import tpu_sc as plsc

N = 10000
E = 320000
D = 128
G = 128
C = 10

NC = 2
NS = 16
NW = NC * NS
CHUNK = 128
ROWS_PW = 320
OUT_ROWS = NW * ROWS_PW
E_PAD = E + 256


def _sc_fold_body(x_hbm, src_hbm, dst_hbm, starts_hbm, out_hbm,
                  acc, rows, idx_s, idx_d, s_vmem, sem):
    c = lax.axis_index("c")
    s = lax.axis_index("s")
    t = c * NS + s
    rowbase = t * ROWS_PW

    zero16 = jnp.zeros((16,), jnp.float32)

    def _zero(j, carry):
        for i in range(D // 16):
            acc[j, pl.ds(i * 16, 16)] = zero16
        return carry

    lax.fori_loop(0, ROWS_PW, _zero, 0, unroll=False)

    pltpu.sync_copy(starts_hbm, s_vmem.at[pl.ds(0, 48)])
    lo = s_vmem[pl.ds(t, 16)][0]
    hi = s_vmem[pl.ds(t + 1, 16)][0]
    base = (lo // 8) * 8
    skip = lo - base
    total = hi - base
    nch = (total + CHUNK - 1) // CHUNK

    def _chunk(k, carry):
        off = base + k * CHUNK
        pltpu.sync_copy(src_hbm.at[pl.ds(off, CHUNK)], idx_s)
        pltpu.sync_copy(dst_hbm.at[pl.ds(off, CHUNK)], idx_d.at[pl.ds(0, CHUNK)])
        pltpu.async_copy(x_hbm.at[idx_s], rows, sem).wait()
        e_lo = jnp.maximum(skip - k * CHUNK, 0)
        e_hi = jnp.minimum(total - k * CHUNK, CHUNK)

        def _fold(e, carry2):
            dl = idx_d[pl.ds(e, 16)][0] - rowbase
            for i in range(D // 16):
                sl = pl.ds(i * 16, 16)
                acc[dl, sl] = acc[dl, sl] + rows[e, sl]
            return carry2

        lax.fori_loop(e_lo, e_hi, _fold, 0, unroll=False)
        return carry

    lax.fori_loop(0, nch, _chunk, 0, unroll=False)
    pltpu.sync_copy(acc, out_hbm.at[pl.ds(rowbase, ROWS_PW)])


@functools.cache
def _sc_fold_kernel():
    return pl.kernel(
        _sc_fold_body,
        out_type=jax.ShapeDtypeStruct((OUT_ROWS, D), jnp.float32),
        mesh=plsc.VectorSubcoreMesh(core_axis_name="c", subcore_axis_name="s",
                                    num_cores=NC, num_subcores=NS),
        scratch_types=[
            pltpu.VMEM((ROWS_PW, D), jnp.float32),
            pltpu.VMEM((CHUNK, D), jnp.float32),
            pltpu.VMEM((CHUNK,), jnp.int32),
            pltpu.VMEM((CHUNK + 16,), jnp.int32),
            pltpu.VMEM((64,), jnp.int32),
            pltpu.SemaphoreType.DMA,
        ],
    )


def _fq_signed(x):
    s = jnp.maximum(jnp.max(jnp.abs(x)), 1e-8) / 127.0
    return jnp.clip(jnp.round(x / s), -128.0, 127.0) * s


def _fq_unsigned(x):
    mn = jnp.min(x)
    mx = jnp.max(x)
    s = jnp.maximum(mx - mn, 1e-8) / 255.0
    zp = jnp.round(-mn / s)
    return (jnp.clip(jnp.round(x / s) + zp, 0.0, 255.0) - zp) * s


def _mlp_pre_bn_body(x_ref, a_ref, w1_ref, b1_ref, w2_ref, b2_ref, o_ref):
    h = x_ref[...] + a_ref[...]
    q = _fq_signed(h)
    y = jnp.dot(q, w1_ref[...], preferred_element_type=jnp.float32) + b1_ref[...]
    y = _fq_signed(y)
    y = _fq_unsigned(jnp.maximum(y, 0.0))
    y = _fq_unsigned(y)
    y = jnp.dot(y, w2_ref[...], preferred_element_type=jnp.float32) + b2_ref[...]
    y = _fq_unsigned(y)
    o_ref[...] = _fq_unsigned(jnp.maximum(y, 0.0))


def _skip_fq_body(y_ref, x_ref, o_ref):
    o_ref[...] = _fq_signed(y_ref[...] + x_ref[...])


def _mlp_call(x, a, w1t, b1, w2t, b2, gam, bet):
    y = pl.pallas_call(
        _mlp_pre_bn_body,
        out_shape=jax.ShapeDtypeStruct((N, D), jnp.float32),
    )(x, a, w1t, b1, w2t, b2)
    mu = jnp.mean(y, axis=0, keepdims=True)
    var = jnp.var(y, axis=0, keepdims=True)
    y = (y - mu) / jnp.sqrt(var + 1e-5) * gam + bet
    return pl.pallas_call(
        _skip_fq_body,
        out_shape=jax.ShapeDtypeStruct((N, D), jnp.float32),
    )(y, x)


_NEG = -3.0e38


def _pool_head_body(x_ref, bid_ref, l1w_ref, l1b_ref, l2w_ref, l2b_ref, o_ref):
    v = x_ref[...]
    ids = bid_ref[...]
    step = 1
    while step < N:
        vs = jnp.concatenate(
            [jnp.full((step, D), _NEG, jnp.float32), v[: N - step]], axis=0)
        is_ = jnp.concatenate(
            [jnp.full((step, 1), -1, jnp.int32), ids[: N - step]], axis=0)
        v = jnp.where(is_ == ids, jnp.maximum(v, vs), v)
        step *= 2
    nxt = jnp.concatenate(
        [ids[1:], jnp.full((1, 1), -1, jnp.int32)], axis=0)
    is_last = (ids != nxt).astype(jnp.float32)
    gi = lax.broadcasted_iota(jnp.int32, (1, G), 1)
    sel = (ids == gi).astype(jnp.float32) * is_last
    pooled = lax.dot_general(sel, v, (((0,), (0,)), ((), ())),
                             precision=lax.Precision.HIGHEST,
                             preferred_element_type=jnp.float32)

    p = _fq_unsigned(pooled)
    y = jnp.dot(p, l1w_ref[...], preferred_element_type=jnp.float32) + l1b_ref[...]
    y = _fq_unsigned(y)
    y = _fq_unsigned(jnp.maximum(y, 0.0))
    y = _fq_unsigned(y)
    z = jnp.dot(y, l2w_ref[...], preferred_element_type=jnp.float32) + l2b_ref[...]
    ci = lax.broadcasted_iota(jnp.int32, (G, D), 1)
    valid = ci < C
    mn = jnp.min(jnp.where(valid, z, 3.0e38))
    mx = jnp.max(jnp.where(valid, z, _NEG))
    s = jnp.maximum(mx - mn, 1e-8) / 255.0
    zp = jnp.round(-mn / s)
    o_ref[...] = (jnp.clip(jnp.round(z / s) + zp, 0.0, 255.0) - zp) * s


def _pool_head_call(x, bid, l1w, l1b, l2w, l2b):
    return pl.pallas_call(
        _pool_head_body,
        out_shape=jax.ShapeDtypeStruct((G, D), jnp.float32),
    )(x, bid, l1w, l1b, l2w, l2b)


def kernel(x, params, edge_index, batch, edge_mask):
    src = edge_index[0]
    dst = edge_index[1]
    owner = jnp.where(edge_mask, dst // ROWS_PW, NW).astype(jnp.int32)
    perm = jnp.argsort(owner, stable=True)
    src_s = jnp.concatenate([src[perm], jnp.zeros((E_PAD - E,), jnp.int32)])
    dst_s = jnp.concatenate([dst[perm], jnp.zeros((E_PAD - E,), jnp.int32)])
    owner_s = owner[perm]
    starts = jnp.searchsorted(
        owner_s, jnp.arange(33, dtype=jnp.int32), side='left').astype(jnp.int32)
    starts = jnp.concatenate([starts, jnp.full((15,), E, jnp.int32)])

    fold = _sc_fold_kernel()
    h = x
    for i in range(3):
        p = params['convs'][i]
        agg = fold(h, src_s, dst_s, starts)
        h = _mlp_call(
            h, agg[:N],
            p['W1'].T, p['b1'].reshape(1, D),
            p['W2'].T, p['b2'].reshape(1, D),
            p['gamma'].reshape(1, D), p['beta'].reshape(1, D),
        )

    l2w = jnp.zeros((D, D), jnp.float32).at[:, :C].set(params['lin2_W'].T)
    l2b = jnp.zeros((1, D), jnp.float32).at[:, :C].set(
        params['lin2_b'].reshape(1, C))
    out = _pool_head_call(
        h, batch.reshape(N, 1),
        params['lin1_W'].T, params['lin1_b'].reshape(1, D),
        l2w, l2b,
    )
    return out[:, :C]


# speedup vs baseline: 3.2087x; 1.0921x over previous
import functools

import jax
import jax.numpy as jnp
from jax import lax
from jax.experimental import pallas as pl
from jax.experimental.pallas import tpu as pltpu
from jax.experimental.pallas import tpu_sc as plsc

N = 10000
E = 320000
D = 128
G = 128
C = 10

NC = 2
NS = 16
NW = NC * NS
CHUNK = 128
ROWS_PW = 320
OUT_ROWS = NW * ROWS_PW
E_PAD = E + 768


def _sc_fold_body(x_hbm, src_hbm, dst_hbm, starts_hbm, out_hbm,
                  acc, rows0, rows1, is0, is1, id0, id1, s_vmem, sem0, sem1):
    c = lax.axis_index("c")
    s = lax.axis_index("s")
    t = c * NS + s
    rowbase = t * ROWS_PW

    zero16 = jnp.zeros((16,), jnp.float32)

    def _zero(j, carry):
        for i in range(D // 16):
            acc[j, pl.ds(i * 16, 16)] = zero16
        return carry

    lax.fori_loop(0, ROWS_PW, _zero, 0, unroll=False)

    pltpu.sync_copy(starts_hbm, s_vmem.at[pl.ds(0, 48)])
    lo = s_vmem[pl.ds(t, 16)][0]
    hi = s_vmem[pl.ds(t + 1, 16)][0]
    base = (lo // 8) * 8
    skip = lo - base
    total = hi - base
    nch = (total + CHUNK - 1) // CHUNK
    npair = (nch + 1) // 2

    def _load_idx(k, isb, idb):
        off = base + k * CHUNK
        pltpu.sync_copy(src_hbm.at[pl.ds(off, CHUNK)], isb)
        pltpu.sync_copy(dst_hbm.at[pl.ds(off, CHUNK)], idb.at[pl.ds(0, CHUNK)])

    def _fold(k, idb, rowsb):
        e_lo = jnp.maximum(skip - k * CHUNK, 0)
        e_hi = jnp.minimum(total - k * CHUNK, CHUNK)

        def _edge(e, carry2):
            dl = idb[pl.ds(e, 16)][0] - rowbase
            for i in range(D // 16):
                sl = pl.ds(i * 16, 16)
                acc[dl, sl] = acc[dl, sl] + rowsb[e, sl]
            return carry2

        lax.fori_loop(e_lo, e_hi, _edge, 0, unroll=False)

    _load_idx(0, is0, id0)
    pltpu.async_copy(x_hbm.at[is0], rows0, sem0)

    def _pair(j, carry):
        k0 = 2 * j
        _load_idx(k0 + 1, is1, id1)
        pltpu.async_copy(x_hbm.at[is1], rows1, sem1)
        pltpu.make_async_copy(x_hbm.at[pl.ds(0, CHUNK)], rows0, sem0).wait()
        _fold(k0, id0, rows0)
        _load_idx(k0 + 2, is0, id0)
        pltpu.async_copy(x_hbm.at[is0], rows0, sem0)
        pltpu.make_async_copy(x_hbm.at[pl.ds(0, CHUNK)], rows1, sem1).wait()
        _fold(k0 + 1, id1, rows1)
        return carry

    lax.fori_loop(0, npair, _pair, 0, unroll=False)
    pltpu.make_async_copy(x_hbm.at[pl.ds(0, CHUNK)], rows0, sem0).wait()
    pltpu.sync_copy(acc, out_hbm.at[pl.ds(rowbase, ROWS_PW)])


@functools.cache
def _sc_fold_kernel():
    return pl.kernel(
        _sc_fold_body,
        out_type=jax.ShapeDtypeStruct((OUT_ROWS, D), jnp.float32),
        mesh=plsc.VectorSubcoreMesh(core_axis_name="c", subcore_axis_name="s",
                                    num_cores=NC, num_subcores=NS),
        scratch_types=[
            pltpu.VMEM((ROWS_PW, D), jnp.float32),
            pltpu.VMEM((CHUNK, D), jnp.float32),
            pltpu.VMEM((CHUNK, D), jnp.float32),
            pltpu.VMEM((CHUNK,), jnp.int32),
            pltpu.VMEM((CHUNK,), jnp.int32),
            pltpu.VMEM((CHUNK + 16,), jnp.int32),
            pltpu.VMEM((CHUNK + 16,), jnp.int32),
            pltpu.VMEM((64,), jnp.int32),
            pltpu.SemaphoreType.DMA,
            pltpu.SemaphoreType.DMA,
        ],
    )


def _fq_signed(x):
    s = jnp.maximum(jnp.max(jnp.abs(x)), 1e-8) / 127.0
    return jnp.clip(jnp.round(x / s), -128.0, 127.0) * s


def _fq_unsigned(x):
    mn = jnp.min(x)
    mx = jnp.max(x)
    s = jnp.maximum(mx - mn, 1e-8) / 255.0
    zp = jnp.round(-mn / s)
    return (jnp.clip(jnp.round(x / s) + zp, 0.0, 255.0) - zp) * s


def _mlp_pre_bn_body(x_ref, a_ref, w1_ref, b1_ref, w2_ref, b2_ref, o_ref):
    h = x_ref[...] + a_ref[...]
    q = _fq_signed(h)
    y = jnp.dot(q, w1_ref[...], preferred_element_type=jnp.float32) + b1_ref[...]
    y = _fq_signed(y)
    y = _fq_unsigned(jnp.maximum(y, 0.0))
    y = _fq_unsigned(y)
    y = jnp.dot(y, w2_ref[...], preferred_element_type=jnp.float32) + b2_ref[...]
    y = _fq_unsigned(y)
    o_ref[...] = _fq_unsigned(jnp.maximum(y, 0.0))


def _skip_fq_body(y_ref, x_ref, o_ref):
    o_ref[...] = _fq_signed(y_ref[...] + x_ref[...])


def _mlp_call(x, a, w1t, b1, w2t, b2, gam, bet):
    y = pl.pallas_call(
        _mlp_pre_bn_body,
        out_shape=jax.ShapeDtypeStruct((N, D), jnp.float32),
    )(x, a, w1t, b1, w2t, b2)
    mu = jnp.mean(y, axis=0, keepdims=True)
    var = jnp.var(y, axis=0, keepdims=True)
    y = (y - mu) / jnp.sqrt(var + 1e-5) * gam + bet
    return pl.pallas_call(
        _skip_fq_body,
        out_shape=jax.ShapeDtypeStruct((N, D), jnp.float32),
    )(y, x)


_NEG = -3.0e38


def _pool_head_body(x_ref, bid_ref, l1w_ref, l1b_ref, l2w_ref, l2b_ref, o_ref):
    v = x_ref[...]
    ids = bid_ref[...]
    step = 1
    while step < N:
        vs = jnp.concatenate(
            [jnp.full((step, D), _NEG, jnp.float32), v[: N - step]], axis=0)
        is_ = jnp.concatenate(
            [jnp.full((step, 1), -1, jnp.int32), ids[: N - step]], axis=0)
        v = jnp.where(is_ == ids, jnp.maximum(v, vs), v)
        step *= 2
    nxt = jnp.concatenate(
        [ids[1:], jnp.full((1, 1), -1, jnp.int32)], axis=0)
    is_last = (ids != nxt).astype(jnp.float32)
    gi = lax.broadcasted_iota(jnp.int32, (1, G), 1)
    sel = (ids == gi).astype(jnp.float32) * is_last
    pooled = lax.dot_general(sel, v, (((0,), (0,)), ((), ())),
                             precision=lax.Precision.HIGHEST,
                             preferred_element_type=jnp.float32)

    p = _fq_unsigned(pooled)
    y = jnp.dot(p, l1w_ref[...], preferred_element_type=jnp.float32) + l1b_ref[...]
    y = _fq_unsigned(y)
    y = _fq_unsigned(jnp.maximum(y, 0.0))
    y = _fq_unsigned(y)
    z = jnp.dot(y, l2w_ref[...], preferred_element_type=jnp.float32) + l2b_ref[...]
    ci = lax.broadcasted_iota(jnp.int32, (G, D), 1)
    valid = ci < C
    mn = jnp.min(jnp.where(valid, z, 3.0e38))
    mx = jnp.max(jnp.where(valid, z, _NEG))
    s = jnp.maximum(mx - mn, 1e-8) / 255.0
    zp = jnp.round(-mn / s)
    o_ref[...] = (jnp.clip(jnp.round(z / s) + zp, 0.0, 255.0) - zp) * s


def _pool_head_call(x, bid, l1w, l1b, l2w, l2b):
    return pl.pallas_call(
        _pool_head_body,
        out_shape=jax.ShapeDtypeStruct((G, D), jnp.float32),
    )(x, bid, l1w, l1b, l2w, l2b)


def kernel(x, params, edge_index, batch, edge_mask):
    src = edge_index[0]
    dst = edge_index[1]
    owner = jnp.where(edge_mask, dst // ROWS_PW, NW).astype(jnp.int32)
    perm = jnp.argsort(owner, stable=True)
    src_s = jnp.concatenate([src[perm], jnp.zeros((E_PAD - E,), jnp.int32)])
    dst_s = jnp.concatenate([dst[perm], jnp.zeros((E_PAD - E,), jnp.int32)])
    owner_s = owner[perm]
    starts = jnp.searchsorted(
        owner_s, jnp.arange(33, dtype=jnp.int32), side='left').astype(jnp.int32)
    starts = jnp.concatenate([starts, jnp.full((15,), E, jnp.int32)])

    fold = _sc_fold_kernel()
    h = x
    for i in range(3):
        p = params['convs'][i]
        agg = fold(h, src_s, dst_s, starts)
        h = _mlp_call(
            h, agg[:N],
            p['W1'].T, p['b1'].reshape(1, D),
            p['W2'].T, p['b2'].reshape(1, D),
            p['gamma'].reshape(1, D), p['beta'].reshape(1, D),
        )

    l2w = jnp.zeros((D, D), jnp.float32).at[:, :C].set(params['lin2_W'].T)
    l2b = jnp.zeros((1, D), jnp.float32).at[:, :C].set(
        params['lin2_b'].reshape(1, C))
    out = _pool_head_call(
        h, batch.reshape(N, 1),
        params['lin1_W'].T, params['lin1_b'].reshape(1, D),
        l2w, l2b,
    )
    return out[:, :C]
